# baseline (speedup 1.0000x reference)
import functools

import jax
import jax.numpy as jnp
from jax import lax
from jax.experimental import pallas as pl
from jax.experimental.pallas import tpu as pltpu
from jax.experimental.pallas import tpu_sc as plsc

_MAX_LEN = 2048
_DIM = 1024
_NC = 2
_NS = 16
_NW = _NC * _NS
_ROWS_PER_W = _MAX_LEN // _NW

_mesh = plsc.VectorSubcoreMesh(core_axis_name="c", subcore_axis_name="s")


@functools.partial(
    pl.kernel,
    mesh=_mesh,
    out_type=jax.ShapeDtypeStruct((_MAX_LEN, _DIM), jnp.float32),
)
def _pos_embed_copy(w_hbm, out_hbm):
    wid = lax.axis_index("s") * _NC + lax.axis_index("c")
    base = wid * _ROWS_PER_W
    pltpu.sync_copy(
        w_hbm.at[pl.ds(base, _ROWS_PER_W)],
        out_hbm.at[pl.ds(base, _ROWS_PER_W)],
    )


def kernel(x, W):
    del x
    return _pos_embed_copy(W)


# speedup vs baseline: 10.6724x; 10.6724x over previous
import functools

import jax
import jax.numpy as jnp
from jax import lax
from jax.experimental import pallas as pl
from jax.experimental.pallas import tpu as pltpu
from jax.experimental.pallas import tpu_sc as plsc

_MAX_LEN = 2048
_DIM = 1024
_NC = 2
_NS = 16
_NW = _NC * _NS
_ROWS_PER_W = _MAX_LEN // _NW
_N_CHUNK = 4
_CH = _ROWS_PER_W // _N_CHUNK

_mesh = plsc.VectorSubcoreMesh(core_axis_name="c", subcore_axis_name="s")


@functools.partial(
    pl.kernel,
    mesh=_mesh,
    out_type=jax.ShapeDtypeStruct((_MAX_LEN, _DIM), jnp.float32),
    scratch_types=[
        pltpu.VMEM((_N_CHUNK, _CH, _DIM), jnp.float32),
        pltpu.SemaphoreType.DMA,
        pltpu.SemaphoreType.DMA,
    ],
)
def _pos_embed_copy(w_hbm, out_hbm, buf, sem_in, sem_out):
    wid = lax.axis_index("s") * _NC + lax.axis_index("c")
    base = wid * _ROWS_PER_W

    copies_in = []
    for i in range(_N_CHUNK):
        c = pltpu.make_async_copy(
            w_hbm.at[pl.ds(base + i * _CH, _CH)], buf.at[i], sem_in
        )
        c.start()
        copies_in.append(c)
    copies_out = []
    for i in range(_N_CHUNK):
        copies_in[i].wait()
        c = pltpu.make_async_copy(
            buf.at[i], out_hbm.at[pl.ds(base + i * _CH, _CH)], sem_out
        )
        c.start()
        copies_out.append(c)
    for c in copies_out:
        c.wait()


def kernel(x, W):
    del x
    return _pos_embed_copy(W)


# speedup vs baseline: 30.1485x; 2.8249x over previous
import jax
import jax.numpy as jnp
from jax.experimental import pallas as pl
from jax.experimental.pallas import tpu as pltpu

_MAX_LEN = 2048
_DIM = 1024
_BLK = 256


def _copy_body(w_ref, o_ref):
    o_ref[...] = w_ref[...]


def kernel(x, W):
    del x
    return pl.pallas_call(
        _copy_body,
        grid=(_MAX_LEN // _BLK,),
        in_specs=[pl.BlockSpec((_BLK, _DIM), lambda i: (i, 0))],
        out_specs=pl.BlockSpec((_BLK, _DIM), lambda i: (i, 0)),
        out_shape=jax.ShapeDtypeStruct((_MAX_LEN, _DIM), jnp.float32),
    )(W)
